# baseline (speedup 1.0000x reference)
import jax
import jax.numpy as jnp
from jax.experimental import pallas as pl

FEAT = 64
SEQ = 1024
NS0, LPS0 = 24, 1
NS1, LPS1 = 7, 24
PERIOD = NS0 * LPS0 * NS1
BN = 32


PROWS = PERIOD * FEAT // 128
SROWS = SEQ * FEAT // 128


def _seasonal_kernel(z_ref, w0_ref, b0_ref, w1_ref, b1_ref, out_ref):
    z = z_ref[...]
    q0 = jnp.dot(z, w0_ref[...], preferred_element_type=jnp.float32) + b0_ref[...]
    q1 = jnp.dot(z, w1_ref[...], preferred_element_type=jnp.float32) + b1_ref[...]
    q0 = q0.reshape(BN, NS0 * FEAT // 128, 128)
    q1 = q1.reshape(BN, NS1, 128)
    tile0 = jnp.concatenate([q0] * NS1, axis=1)
    rep1 = jnp.repeat(q1, NS0 * FEAT // 128, axis=1)
    base = tile0 + rep1
    nfull = SROWS // PROWS
    for i in range(nfull):
        out_ref[:, i * PROWS:(i + 1) * PROWS, :] = base
    tail = SROWS - nfull * PROWS
    if tail:
        out_ref[:, nfull * PROWS:, :] = base[:, :tail, :]


def kernel(z, W0, b0, W1, b1):
    N, LATENT = z.shape
    W0r = W0.reshape(LATENT, FEAT, NS0).transpose(0, 2, 1).reshape(LATENT, FEAT * NS0)
    b0r = b0.reshape(FEAT, NS0).transpose(1, 0).reshape(1, FEAT * NS0)
    W1r = W1.reshape(LATENT, FEAT, NS1).transpose(0, 2, 1)
    W1r = jnp.concatenate([W1r, W1r], axis=2).reshape(LATENT, NS1 * 128)
    b1r = b1.reshape(FEAT, NS1).transpose(1, 0)
    b1r = jnp.concatenate([b1r, b1r], axis=1).reshape(1, NS1 * 128)

    grid = (N // BN,)
    out = pl.pallas_call(
        _seasonal_kernel,
        grid=grid,
        in_specs=[
            pl.BlockSpec((BN, LATENT), lambda i: (i, 0)),
            pl.BlockSpec((LATENT, FEAT * NS0), lambda i: (0, 0)),
            pl.BlockSpec((1, FEAT * NS0), lambda i: (0, 0)),
            pl.BlockSpec((LATENT, NS1 * 128), lambda i: (0, 0)),
            pl.BlockSpec((1, NS1 * 128), lambda i: (0, 0)),
        ],
        out_specs=pl.BlockSpec((BN, SROWS, 128), lambda i: (i, 0, 0)),
        out_shape=jax.ShapeDtypeStruct((N, SROWS, 128), jnp.float32),
    )(z, W0r, b0r, W1r, b1r)
    return out.reshape(N, SEQ, FEAT)


# speedup vs baseline: 1.0333x; 1.0333x over previous
import jax
import jax.numpy as jnp
from jax.experimental import pallas as pl

FEAT = 64
SEQ = 1024
NS0, LPS0 = 24, 1
NS1, LPS1 = 7, 24
PERIOD = NS0 * LPS0 * NS1
BN = 32


def _seasonal_kernel(z_ref, w0_ref, b0_ref, w1_ref, b1_ref, out_ref):
    z = z_ref[...]
    p0 = jnp.dot(z, w0_ref[...], preferred_element_type=jnp.float32) + b0_ref[...]
    p1 = jnp.dot(z, w1_ref[...], preferred_element_type=jnp.float32) + b1_ref[...]
    p0 = p0.reshape(BN, NS0, FEAT)
    p1 = p1.reshape(BN, NS1, FEAT)
    tile0 = jnp.concatenate([p0] * NS1, axis=1)
    rep1 = jnp.repeat(p1, LPS1, axis=1)
    base = tile0 + rep1
    nfull = SEQ // PERIOD
    for i in range(nfull):
        out_ref[:, i * PERIOD:(i + 1) * PERIOD, :] = base
    tail = SEQ - nfull * PERIOD
    if tail:
        out_ref[:, nfull * PERIOD:, :] = base[:, :tail, :]


def kernel(z, W0, b0, W1, b1):
    N, LATENT = z.shape
    W0r = W0.reshape(LATENT, FEAT, NS0).transpose(0, 2, 1).reshape(LATENT, FEAT * NS0)
    b0r = b0.reshape(FEAT, NS0).transpose(1, 0).reshape(1, FEAT * NS0)
    W1r = W1.reshape(LATENT, FEAT, NS1).transpose(0, 2, 1).reshape(LATENT, FEAT * NS1)
    b1r = b1.reshape(FEAT, NS1).transpose(1, 0).reshape(1, FEAT * NS1)

    grid = (N // BN,)
    return pl.pallas_call(
        _seasonal_kernel,
        grid=grid,
        in_specs=[
            pl.BlockSpec((BN, LATENT), lambda i: (i, 0)),
            pl.BlockSpec((LATENT, FEAT * NS0), lambda i: (0, 0)),
            pl.BlockSpec((1, FEAT * NS0), lambda i: (0, 0)),
            pl.BlockSpec((LATENT, FEAT * NS1), lambda i: (0, 0)),
            pl.BlockSpec((1, FEAT * NS1), lambda i: (0, 0)),
        ],
        out_specs=pl.BlockSpec((BN, SEQ, FEAT), lambda i: (i, 0, 0)),
        out_shape=jax.ShapeDtypeStruct((N, SEQ, FEAT), jnp.float32),
    )(z, W0r, b0r, W1r, b1r)


# speedup vs baseline: 1.4628x; 1.4156x over previous
import jax
import jax.numpy as jnp
from jax.experimental import pallas as pl

FEAT = 64
SEQ = 1024
NS0 = 24
NS1 = 7
LPS1 = 24
BN = 16


def _seasonal_kernel(z_ref, w0_ref, b0_ref, w1_ref, b1_ref, out_ref):
    z = z_ref[...]
    p0 = jnp.dot(z, w0_ref[...], preferred_element_type=jnp.float32) + b0_ref[...]
    p1 = jnp.dot(z, w1_ref[...], preferred_element_type=jnp.float32) + b1_ref[...]
    p0 = p0.reshape(BN, FEAT, NS0)
    p1 = p1.reshape(BN, FEAT, NS1)
    per = (jnp.concatenate([p0] * NS1, axis=2)
           + jnp.repeat(p1, LPS1, axis=2))
    tail = SEQ - (SEQ // (NS0 * NS1)) * (NS0 * NS1)
    vals = jnp.concatenate([per] * (SEQ // (NS0 * NS1)) + [per[:, :, :tail]],
                           axis=2)
    out_ref[...] = vals


def kernel(z, W0, b0, W1, b1):
    N, LATENT = z.shape
    b0r = b0.reshape(1, FEAT * NS0)
    b1r = b1.reshape(1, FEAT * NS1)
    grid = (N // BN,)
    vals = pl.pallas_call(
        _seasonal_kernel,
        grid=grid,
        in_specs=[
            pl.BlockSpec((BN, LATENT), lambda i: (i, 0)),
            pl.BlockSpec((LATENT, FEAT * NS0), lambda i: (0, 0)),
            pl.BlockSpec((1, FEAT * NS0), lambda i: (0, 0)),
            pl.BlockSpec((LATENT, FEAT * NS1), lambda i: (0, 0)),
            pl.BlockSpec((1, FEAT * NS1), lambda i: (0, 0)),
        ],
        out_specs=pl.BlockSpec((BN, FEAT, SEQ), lambda i: (i, 0, 0)),
        out_shape=jax.ShapeDtypeStruct((N, FEAT, SEQ), jnp.float32),
    )(z, W0, b0r, W1, b1r)
    return jnp.transpose(vals, (0, 2, 1))


# speedup vs baseline: 2.1694x; 1.4831x over previous
import jax
import jax.numpy as jnp
from jax.experimental import pallas as pl

FEAT = 64
SEQ = 1024
NS0 = 24
NS1 = 7
LPS1 = 24
BN = 16


def _seasonal_kernel(z_ref, w0_ref, b0_ref, w1_ref, b1_ref, out_ref):
    z = z_ref[...]
    p0 = jnp.dot(z, w0_ref[...], preferred_element_type=jnp.float32) + b0_ref[...]
    p1 = jnp.dot(z, w1_ref[...], preferred_element_type=jnp.float32) + b1_ref[...]
    p0 = p0.reshape(BN, FEAT, NS0)
    p1 = p1.reshape(BN, FEAT, NS1)
    base = jnp.concatenate(
        [p0 + p1[:, :, j:j + 1] for j in range(NS1)], axis=2)
    out_ref[...] = jnp.concatenate(
        [base] * (SEQ // (NS0 * NS1)) + [base[:, :, :SEQ % (NS0 * NS1)]],
        axis=2)


def kernel(z, W0, b0, W1, b1):
    N, LATENT = z.shape
    b0r = b0.reshape(1, FEAT * NS0)
    b1r = b1.reshape(1, FEAT * NS1)
    grid = (N // BN,)
    vals = pl.pallas_call(
        _seasonal_kernel,
        grid=grid,
        in_specs=[
            pl.BlockSpec((BN, LATENT), lambda i: (i, 0)),
            pl.BlockSpec((LATENT, FEAT * NS0), lambda i: (0, 0)),
            pl.BlockSpec((1, FEAT * NS0), lambda i: (0, 0)),
            pl.BlockSpec((LATENT, FEAT * NS1), lambda i: (0, 0)),
            pl.BlockSpec((1, FEAT * NS1), lambda i: (0, 0)),
        ],
        out_specs=pl.BlockSpec((BN, FEAT, SEQ), lambda i: (i, 0, 0)),
        out_shape=jax.ShapeDtypeStruct((N, FEAT, SEQ), jnp.float32),
    )(z, W0, b0r, W1, b1r)
    return jnp.transpose(vals, (0, 2, 1))


# speedup vs baseline: 2.6447x; 1.2191x over previous
import jax
import jax.numpy as jnp
from jax.experimental import pallas as pl

FEAT = 64
SEQ = 1024
NS0 = 24
NS1 = 7
LPS1 = 24
BN = 16


def _tables_kernel(z_ref, w0_ref, b0_ref, w1_ref, b1_ref, p0_ref, p1_ref):
    z = z_ref[...]
    p0_ref[...] = (
        jnp.dot(z, w0_ref[...], preferred_element_type=jnp.float32) + b0_ref[...]
    )
    p1_ref[...] = (
        jnp.dot(z, w1_ref[...], preferred_element_type=jnp.float32) + b1_ref[...]
    )


def _expand_kernel(p0_ref, p1_ref, out_ref):
    t0 = jax.lax.broadcasted_iota(jnp.int32, (NS0, SEQ), 1)
    s0 = jax.lax.broadcasted_iota(jnp.int32, (NS0, SEQ), 0)
    g0 = ((t0 % NS0) == s0).astype(jnp.float32)
    t1 = jax.lax.broadcasted_iota(jnp.int32, (NS1, SEQ), 1)
    s1 = jax.lax.broadcasted_iota(jnp.int32, (NS1, SEQ), 0)
    g1 = (((t1 // LPS1) % NS1) == s1).astype(jnp.float32)
    out_ref[...] = jnp.dot(
        p0_ref[...], g0, preferred_element_type=jnp.float32
    ) + jnp.dot(p1_ref[...], g1, preferred_element_type=jnp.float32)


def kernel(z, W0, b0, W1, b1):
    N, LATENT = z.shape
    b0r = b0.reshape(1, FEAT * NS0)
    b1r = b1.reshape(1, FEAT * NS1)
    p0, p1 = pl.pallas_call(
        _tables_kernel,
        grid=(1,),
        in_specs=[
            pl.BlockSpec((N, LATENT), lambda i: (0, 0)),
            pl.BlockSpec((LATENT, FEAT * NS0), lambda i: (0, 0)),
            pl.BlockSpec((1, FEAT * NS0), lambda i: (0, 0)),
            pl.BlockSpec((LATENT, FEAT * NS1), lambda i: (0, 0)),
            pl.BlockSpec((1, FEAT * NS1), lambda i: (0, 0)),
        ],
        out_specs=[
            pl.BlockSpec((N, FEAT * NS0), lambda i: (0, 0)),
            pl.BlockSpec((N, FEAT * NS1), lambda i: (0, 0)),
        ],
        out_shape=[
            jax.ShapeDtypeStruct((N, FEAT * NS0), jnp.float32),
            jax.ShapeDtypeStruct((N, FEAT * NS1), jnp.float32),
        ],
    )(z, W0, b0r, W1, b1r)
    p0r = p0.reshape(N * FEAT, NS0)
    p1r = p1.reshape(N * FEAT, NS1)
    vals = pl.pallas_call(
        _expand_kernel,
        grid=(N // BN,),
        in_specs=[
            pl.BlockSpec((BN * FEAT, NS0), lambda i: (i, 0)),
            pl.BlockSpec((BN * FEAT, NS1), lambda i: (i, 0)),
        ],
        out_specs=pl.BlockSpec((BN * FEAT, SEQ), lambda i: (i, 0)),
        out_shape=jax.ShapeDtypeStruct((N * FEAT, SEQ), jnp.float32),
    )(p0r, p1r)
    return jnp.transpose(vals.reshape(N, FEAT, SEQ), (0, 2, 1))
